# baseline (speedup 1.0000x reference)
import functools

import jax
import jax.numpy as jnp
import numpy as np
from jax import lax
from jax.experimental import pallas as pl
from jax.experimental.pallas import tpu as pltpu
from jax.experimental.pallas import tpu_sc as plsc

F = 26
CARD = 100000
D = 32
LANES = 16
NUM_CORES = 2
NUM_SUBCORES = 16
NUM_WORKERS = NUM_CORES * NUM_SUBCORES
W = 416


@functools.lru_cache(maxsize=None)
def _build(total_n):
    chunk = total_n // NUM_WORKERS
    n_win = chunk // W
    mesh = plsc.VectorSubcoreMesh(core_axis_name="c", subcore_axis_name="s")

    @functools.partial(
        pl.kernel,
        out_type=jax.ShapeDtypeStruct((total_n, D), jnp.float32),
        mesh=mesh,
        compiler_params=pltpu.CompilerParams(use_tc_tiling_on_sc=False),
        scratch_types=[
            pltpu.VMEM((W,), jnp.int32),
            pltpu.VMEM((W,), jnp.int32),
            pltpu.VMEM((F, D), jnp.float32),
            pltpu.VMEM((W, D), jnp.float32),
            pltpu.SemaphoreType.DMA,
        ],
    )
    def k(table_hbm, x_hbm, offs_hbm, bias_hbm, out_hbm,
          idx_v, offs_v, bias_v, rows_v, sem):
        wid = lax.axis_index("s") * NUM_CORES + lax.axis_index("c")
        pltpu.sync_copy(offs_hbm, offs_v)
        pltpu.sync_copy(bias_hbm, bias_v)

        @pl.loop(0, n_win)
        def _window(w):
            base = wid * chunk + w * W
            pltpu.sync_copy(x_hbm.at[pl.ds(base, W)], idx_v)

            @pl.loop(0, W // LANES)
            def _offset_add(i):
                s = pl.ds(i * LANES, LANES)
                idx_v[s] = idx_v[s] + offs_v[s]

            pltpu.async_copy(table_hbm.at[idx_v], rows_v, sem).wait()

            @pl.loop(0, F)
            def _bias_field(f):
                b0 = bias_v[f, pl.ds(0, LANES)]
                b1 = bias_v[f, pl.ds(LANES, LANES)]

                @pl.loop(0, W // F)
                def _bias_row(g):
                    r = g * F + f
                    rows_v[r, pl.ds(0, LANES)] += b0
                    rows_v[r, pl.ds(LANES, LANES)] += b1

            pltpu.sync_copy(rows_v, out_hbm.at[pl.ds(base, W)])

    return k


_OFFS_TILE = np.tile(np.arange(F, dtype=np.int32) * CARD, W // F)


def kernel(x, table, bias):
    b, f = x.shape
    n = b * f
    x_flat = x.reshape(n)
    offs = jnp.asarray(_OFFS_TILE)
    out = _build(n)(table, x_flat, offs, bias)
    return out.reshape(b, f, D)


# speedup vs baseline: 1.3107x; 1.3107x over previous
import functools

import jax
import jax.numpy as jnp
import numpy as np
from jax import lax
from jax.experimental import pallas as pl
from jax.experimental.pallas import tpu as pltpu
from jax.experimental.pallas import tpu_sc as plsc

F = 26
CARD = 100000
D = 32
DP = 128
LANES = 16
NUM_CORES = 2
NUM_SUBCORES = 16
NUM_WORKERS = NUM_CORES * NUM_SUBCORES
W = 416


@functools.lru_cache(maxsize=None)
def _build(total_n, v):
    chunk = total_n // NUM_WORKERS
    n_win = chunk // W
    mesh = plsc.VectorSubcoreMesh(core_axis_name="c", subcore_axis_name="s")

    @functools.partial(
        pl.kernel,
        out_type=jax.ShapeDtypeStruct((total_n * D,), jnp.float32),
        mesh=mesh,
        scratch_types=[
            pltpu.VMEM((W,), jnp.int32),
            pltpu.VMEM((W,), jnp.int32),
            pltpu.VMEM((F * D,), jnp.float32),
            pltpu.VMEM((W, DP), jnp.float32),
            pltpu.VMEM((W * D,), jnp.float32),
            pltpu.SemaphoreType.DMA,
        ],
    )
    def k(table_hbm, x_hbm, offs_hbm, bias_hbm, out_hbm,
          idx_v, offs_v, bias_v, rows_v, out_v, sem):
        wid = lax.axis_index("s") * NUM_CORES + lax.axis_index("c")
        pltpu.sync_copy(offs_hbm, offs_v)
        pltpu.sync_copy(bias_hbm, bias_v)

        @pl.loop(0, n_win)
        def _window(w):
            base = wid * chunk + w * W
            pltpu.sync_copy(x_hbm.at[pl.ds(base, W)], idx_v)

            @pl.loop(0, W // LANES)
            def _offset_add(i):
                s = pl.ds(i * LANES, LANES)
                idx_v[s] = idx_v[s] + offs_v[s]

            pltpu.async_copy(table_hbm.at[idx_v], rows_v, sem).wait()

            @pl.loop(0, F)
            def _bias_field(f):
                b0 = bias_v[pl.ds(f * D, LANES)]
                b1 = bias_v[pl.ds(f * D + LANES, LANES)]

                @pl.loop(0, W // F)
                def _bias_row(g):
                    r = g * F + f
                    out_v[pl.ds(r * D, LANES)] = (
                        rows_v[r, pl.ds(0, LANES)] + b0)
                    out_v[pl.ds(r * D + LANES, LANES)] = (
                        rows_v[r, pl.ds(LANES, LANES)] + b1)

            pltpu.sync_copy(out_v, out_hbm.at[pl.ds(base * D, W * D)])

    return k


_OFFS_TILE = np.tile(np.arange(F, dtype=np.int32) * CARD, W // F)

_TC_LANES = 4096


def _transpose_body(x_ref, o_ref):
    o_ref[:, 0:D] = jnp.transpose(x_ref[...], (1, 0))


@functools.lru_cache(maxsize=None)
def _build_transpose(v):
    grid = (v + _TC_LANES - 1) // _TC_LANES
    return pl.pallas_call(
        _transpose_body,
        grid=(grid,),
        in_specs=[pl.BlockSpec((D, _TC_LANES), lambda i: (0, i))],
        out_specs=pl.BlockSpec((_TC_LANES, DP), lambda i: (i, 0)),
        out_shape=jax.ShapeDtypeStruct((v, DP), jnp.float32),
        compiler_params=pltpu.CompilerParams(
            dimension_semantics=("parallel",)),
    )


def kernel(x, table, bias):
    b, f = x.shape
    n = b * f
    v = table.shape[0]
    x_flat = x.reshape(n)
    offs = jnp.asarray(_OFFS_TILE)
    table_p = _build_transpose(v)(table.T)
    out = _build(n, v)(table_p, x_flat, offs, bias.reshape(-1))
    return out.reshape(b, f, D)


# speedup vs baseline: 1.3706x; 1.0457x over previous
import functools

import jax
import jax.numpy as jnp
import numpy as np
from jax import lax
from jax.experimental import pallas as pl
from jax.experimental.pallas import tpu as pltpu
from jax.experimental.pallas import tpu_sc as plsc

F = 26
CARD = 100000
D = 32
DP = 128
LANES = 16
NUM_CORES = 2
NUM_SUBCORES = 16
NUM_WORKERS = NUM_CORES * NUM_SUBCORES
W = 416


@functools.lru_cache(maxsize=None)
def _build(total_n, v):
    chunk = total_n // NUM_WORKERS
    n_win = chunk // W
    mesh = plsc.VectorSubcoreMesh(core_axis_name="c", subcore_axis_name="s")

    n_b = total_n // F
    b_chunk = chunk // F
    b_win = W // F

    @functools.partial(
        pl.kernel,
        out_type=jax.ShapeDtypeStruct((n_b, 32, DP), jnp.float32),
        mesh=mesh,
        scratch_types=[
            pltpu.VMEM((W,), jnp.int32),
            pltpu.VMEM((W,), jnp.int32),
            pltpu.VMEM((F * D,), jnp.float32),
            pltpu.VMEM((W + 16, DP), jnp.float32),
            pltpu.SemaphoreType.DMA,
        ],
    )
    def k(table_hbm, x_hbm, offs_hbm, bias_hbm, out_hbm,
          idx_v, offs_v, bias_v, rows_v, sem):
        wid = lax.axis_index("s") * NUM_CORES + lax.axis_index("c")
        pltpu.sync_copy(offs_hbm, offs_v)
        pltpu.sync_copy(bias_hbm, bias_v)

        @pl.loop(0, n_win)
        def _window(w):
            base = wid * chunk + w * W
            b_base = wid * b_chunk + w * b_win
            pltpu.sync_copy(x_hbm.at[pl.ds(base, W)], idx_v)

            @pl.loop(0, W // LANES)
            def _offset_add(i):
                s = pl.ds(i * LANES, LANES)
                idx_v[s] = idx_v[s] + offs_v[s]

            pltpu.async_copy(table_hbm.at[idx_v],
                             rows_v.at[pl.ds(0, W)], sem).wait()

            @pl.loop(0, F)
            def _bias_field(f):
                b0 = bias_v[pl.ds(f * D, LANES)]
                b1 = bias_v[pl.ds(f * D + LANES, LANES)]

                @pl.loop(0, b_win)
                def _bias_row(g):
                    r = g * F + f
                    rows_v[r, pl.ds(0, LANES)] += b0
                    rows_v[r, pl.ds(LANES, LANES)] += b1

            @pl.loop(0, b_win)
            def _store_b(g):
                pltpu.sync_copy(rows_v.at[pl.ds(g * F, 32)],
                                out_hbm.at[b_base + g])

    return k


_OFFS_TILE = np.tile(np.arange(F, dtype=np.int32) * CARD, W // F)

_TC_LANES = 4096


def _transpose_body(x_ref, o_ref):
    o_ref[:, 0:D] = jnp.transpose(x_ref[...], (1, 0))


@functools.lru_cache(maxsize=None)
def _build_transpose(v):
    grid = (v + _TC_LANES - 1) // _TC_LANES
    return pl.pallas_call(
        _transpose_body,
        grid=(grid,),
        in_specs=[pl.BlockSpec((D, _TC_LANES), lambda i: (0, i))],
        out_specs=pl.BlockSpec((_TC_LANES, DP), lambda i: (i, 0)),
        out_shape=jax.ShapeDtypeStruct((v, DP), jnp.float32),
        compiler_params=pltpu.CompilerParams(
            dimension_semantics=("parallel",)),
    )


def kernel(x, table, bias):
    b, f = x.shape
    n = b * f
    v = table.shape[0]
    x_flat = x.reshape(n)
    offs = jnp.asarray(_OFFS_TILE)
    table_p = _build_transpose(v)(table.T)
    out3 = _build(n, v)(table_p, x_flat, offs, bias.reshape(-1))
    return out3[:, :f, :D]


# speedup vs baseline: 1.5970x; 1.1652x over previous
import functools

import jax
import jax.numpy as jnp
import numpy as np
from jax import lax
from jax.experimental import pallas as pl
from jax.experimental.pallas import tpu as pltpu
from jax.experimental.pallas import tpu_sc as plsc

F = 26
CARD = 100000
D = 32
DP = 128
LANES = 16
NUM_CORES = 2
NUM_SUBCORES = 16
NUM_WORKERS = NUM_CORES * NUM_SUBCORES
W = 416


@functools.lru_cache(maxsize=None)
def _build(total_n, v):
    chunk = total_n // NUM_WORKERS
    n_win = chunk // W
    mesh = plsc.VectorSubcoreMesh(core_axis_name="c", subcore_axis_name="s")

    n_b = total_n // F
    b_chunk = chunk // F
    b_win = W // F

    @functools.partial(
        pl.kernel,
        out_type=jax.ShapeDtypeStruct((n_b, 32, DP), jnp.float32),
        mesh=mesh,
        scratch_types=[
            pltpu.VMEM((W,), jnp.int32),
            pltpu.VMEM((W,), jnp.int32),
            pltpu.VMEM((F * D,), jnp.float32),
            pltpu.VMEM((W + 16, DP), jnp.float32),
            pltpu.SemaphoreType.DMA,
        ],
    )
    def k(table_hbm, x_hbm, offs_hbm, bias_hbm, out_hbm,
          idx_v, offs_v, bias_v, rows_v, sem):
        wid = lax.axis_index("s") * NUM_CORES + lax.axis_index("c")
        pltpu.sync_copy(offs_hbm, offs_v)
        pltpu.sync_copy(bias_hbm, bias_v)

        @pl.loop(0, n_win)
        def _window(w):
            base = wid * chunk + w * W
            b_base = wid * b_chunk + w * b_win
            pltpu.sync_copy(x_hbm.at[pl.ds(base, W)], idx_v)

            @pl.loop(0, W // LANES)
            def _offset_add(i):
                s = pl.ds(i * LANES, LANES)
                idx_v[s] = idx_v[s] + offs_v[s]

            pltpu.async_copy(table_hbm.at[idx_v],
                             rows_v.at[pl.ds(0, W)], sem).wait()

            @pl.loop(0, F)
            def _bias_field(f):
                b0 = bias_v[pl.ds(f * D, LANES)]
                b1 = bias_v[pl.ds(f * D + LANES, LANES)]

                @pl.loop(0, b_win)
                def _bias_row(g):
                    r = g * F + f
                    rows_v[r, pl.ds(0, LANES)] += b0
                    rows_v[r, pl.ds(LANES, LANES)] += b1

            @pl.loop(0, b_win)
            def _store_b(g):
                pltpu.sync_copy(rows_v.at[pl.ds(g * F, 32)],
                                out_hbm.at[b_base + g])

    return k


_OFFS_TILE = np.tile(np.arange(F, dtype=np.int32) * CARD, W // F)

_TC_LANES = 8192


def _transpose_body(x_ref, o_ref):
    o_ref[:, 0:D] = jnp.transpose(x_ref[...], (1, 0))


@functools.lru_cache(maxsize=None)
def _build_transpose(v):
    grid = (v + _TC_LANES - 1) // _TC_LANES
    return pl.pallas_call(
        _transpose_body,
        grid=(grid,),
        in_specs=[pl.BlockSpec((D, _TC_LANES), lambda i: (0, i))],
        out_specs=pl.BlockSpec((_TC_LANES, DP), lambda i: (i, 0)),
        out_shape=jax.ShapeDtypeStruct((v, DP), jnp.float32),
        compiler_params=pltpu.CompilerParams(
            dimension_semantics=("parallel",)),
    )


def kernel(x, table, bias):
    b, f = x.shape
    n = b * f
    v = table.shape[0]
    x_flat = x.reshape(n)
    offs = jnp.asarray(_OFFS_TILE)
    table_p = _build_transpose(v)(table.T)
    out3 = _build(n, v)(table_p, x_flat, offs, bias.reshape(-1))
    return out3[:, :f, :D]


# speedup vs baseline: 1.7331x; 1.0852x over previous
import functools

import jax
import jax.numpy as jnp
import numpy as np
from jax import lax
from jax.experimental import pallas as pl
from jax.experimental.pallas import tpu as pltpu
from jax.experimental.pallas import tpu_sc as plsc

F = 26
CARD = 100000
D = 32
DP = 128
LANES = 16
NUM_CORES = 2
NUM_SUBCORES = 16
NUM_WORKERS = NUM_CORES * NUM_SUBCORES
W = 416


@functools.lru_cache(maxsize=None)
def _build(total_n, v):
    chunk = total_n // NUM_WORKERS
    n_win = chunk // W
    mesh = plsc.VectorSubcoreMesh(core_axis_name="c", subcore_axis_name="s")

    n_b = total_n // F
    b_chunk = chunk // F
    b_win = W // F

    @functools.partial(
        pl.kernel,
        out_type=jax.ShapeDtypeStruct((n_b, 32, DP), jnp.float32),
        mesh=mesh,
        scratch_types=[
            pltpu.VMEM((W,), jnp.int32),
            pltpu.VMEM((W,), jnp.int32),
            pltpu.VMEM((F * D,), jnp.float32),
            pltpu.VMEM((W + 16, DP), jnp.float32),
            pltpu.SemaphoreType.DMA,
        ],
    )
    def k(table_hbm, x_hbm, offs_hbm, bias_hbm, out_hbm,
          idx_v, offs_v, bias_v, rows_v, sem):
        wid = lax.axis_index("s") * NUM_CORES + lax.axis_index("c")
        pltpu.sync_copy(offs_hbm, offs_v)
        pltpu.sync_copy(bias_hbm, bias_v)

        @pl.loop(0, n_win)
        def _window(w):
            base = wid * chunk + w * W
            b_base = wid * b_chunk + w * b_win
            pltpu.sync_copy(x_hbm.at[pl.ds(base, W)], idx_v)

            @pl.loop(0, W // LANES)
            def _offset_add(i):
                s = pl.ds(i * LANES, LANES)
                idx_v[s] = idx_v[s] + offs_v[s]

            pltpu.async_copy(table_hbm.at[idx_v],
                             rows_v.at[pl.ds(0, W)], sem).wait()

            @pl.loop(0, F)
            def _bias_field(f):
                b0 = bias_v[pl.ds(f * D, LANES)]
                b1 = bias_v[pl.ds(f * D + LANES, LANES)]

                @pl.loop(0, b_win)
                def _bias_row(g):
                    r = g * F + f
                    rows_v[r, pl.ds(0, LANES)] += b0
                    rows_v[r, pl.ds(LANES, LANES)] += b1

            @pl.loop(0, b_win)
            def _store_b(g):
                pltpu.sync_copy(rows_v.at[pl.ds(g * F, 32)],
                                out_hbm.at[b_base + g])

    return k


_OFFS_TILE = np.tile(np.arange(F, dtype=np.int32) * CARD, W // F)

_TC_LANES = 16384


def _transpose_body(x_ref, o_ref):
    o_ref[:, 0:D] = jnp.transpose(x_ref[...], (1, 0))


@functools.lru_cache(maxsize=None)
def _build_transpose(v):
    grid = (v + _TC_LANES - 1) // _TC_LANES
    return pl.pallas_call(
        _transpose_body,
        grid=(grid,),
        in_specs=[pl.BlockSpec((D, _TC_LANES), lambda i: (0, i))],
        out_specs=pl.BlockSpec((_TC_LANES, DP), lambda i: (i, 0)),
        out_shape=jax.ShapeDtypeStruct((v, DP), jnp.float32),
        compiler_params=pltpu.CompilerParams(
            dimension_semantics=("parallel",)),
    )


def kernel(x, table, bias):
    b, f = x.shape
    n = b * f
    v = table.shape[0]
    x_flat = x.reshape(n)
    offs = jnp.asarray(_OFFS_TILE)
    table_p = _build_transpose(v)(table.T)
    out3 = _build(n, v)(table_p, x_flat, offs, bias.reshape(-1))
    return out3[:, :f, :D]


# speedup vs baseline: 1.7619x; 1.0167x over previous
import functools

import jax
import jax.numpy as jnp
import numpy as np
from jax import lax
from jax.experimental import pallas as pl
from jax.experimental.pallas import tpu as pltpu
from jax.experimental.pallas import tpu_sc as plsc

F = 26
CARD = 100000
D = 32
DP = 128
LANES = 16
NUM_CORES = 2
NUM_SUBCORES = 16
NUM_WORKERS = NUM_CORES * NUM_SUBCORES
W = 416


@functools.lru_cache(maxsize=None)
def _build(total_n, v):
    chunk = total_n // NUM_WORKERS
    n_win = chunk // W
    mesh = plsc.VectorSubcoreMesh(core_axis_name="c", subcore_axis_name="s")

    n_b = total_n // F
    b_chunk = chunk // F
    b_win = W // F

    @functools.partial(
        pl.kernel,
        out_type=jax.ShapeDtypeStruct((n_b, 32, DP), jnp.float32),
        mesh=mesh,
        scratch_types=[
            pltpu.VMEM((W,), jnp.int32),
            pltpu.VMEM((W,), jnp.int32),
            pltpu.VMEM((F * D,), jnp.float32),
            pltpu.VMEM((W + 16, DP), jnp.float32),
            pltpu.SemaphoreType.DMA,
        ],
    )
    def k(table_hbm, x_hbm, offs_hbm, bias_hbm, out_hbm,
          idx_v, offs_v, bias_v, rows_v, sem):
        wid = lax.axis_index("s") * NUM_CORES + lax.axis_index("c")
        pltpu.sync_copy(offs_hbm, offs_v)
        pltpu.sync_copy(bias_hbm, bias_v)

        @pl.loop(0, n_win)
        def _window(w):
            base = wid * chunk + w * W
            b_base = wid * b_chunk + w * b_win
            pltpu.sync_copy(x_hbm.at[pl.ds(base, W)], idx_v)

            @pl.loop(0, W // LANES)
            def _offset_add(i):
                s = pl.ds(i * LANES, LANES)
                idx_v[s] = idx_v[s] + offs_v[s]

            pltpu.async_copy(table_hbm.at[idx_v],
                             rows_v.at[pl.ds(0, W)], sem).wait()

            @pl.loop(0, F)
            def _bias_field(f):
                b0 = bias_v[pl.ds(f * D, LANES)]
                b1 = bias_v[pl.ds(f * D + LANES, LANES)]

                @pl.loop(0, b_win)
                def _bias_row(g):
                    r = g * F + f
                    rows_v[r, pl.ds(0, LANES)] += b0
                    rows_v[r, pl.ds(LANES, LANES)] += b1

            @pl.loop(0, b_win)
            def _store_b(g):
                pltpu.sync_copy(rows_v.at[pl.ds(g * F, 32)],
                                out_hbm.at[b_base + g])

    return k


_OFFS_TILE = np.tile(np.arange(F, dtype=np.int32) * CARD, W // F)

_TC_LANES = 32768


def _transpose_body(x_ref, o_ref):
    o_ref[:, 0:D] = jnp.transpose(x_ref[...], (1, 0))


@functools.lru_cache(maxsize=None)
def _build_transpose(v):
    grid = (v + _TC_LANES - 1) // _TC_LANES
    return pl.pallas_call(
        _transpose_body,
        grid=(grid,),
        in_specs=[pl.BlockSpec((D, _TC_LANES), lambda i: (0, i))],
        out_specs=pl.BlockSpec((_TC_LANES, DP), lambda i: (i, 0)),
        out_shape=jax.ShapeDtypeStruct((v, DP), jnp.float32),
        compiler_params=pltpu.CompilerParams(
            dimension_semantics=("parallel",)),
    )


def kernel(x, table, bias):
    b, f = x.shape
    n = b * f
    v = table.shape[0]
    x_flat = x.reshape(n)
    offs = jnp.asarray(_OFFS_TILE)
    table_p = _build_transpose(v)(table.T)
    out3 = _build(n, v)(table_p, x_flat, offs, bias.reshape(-1))
    return out3[:, :f, :D]


# speedup vs baseline: 1.9294x; 1.0951x over previous
import functools

import jax
import jax.numpy as jnp
import numpy as np
from jax import lax
from jax.experimental import pallas as pl
from jax.experimental.pallas import tpu as pltpu
from jax.experimental.pallas import tpu_sc as plsc

F = 26
CARD = 100000
D = 32
DP = 128
LANES = 16
NUM_CORES = 2
NUM_SUBCORES = 16
NUM_WORKERS = NUM_CORES * NUM_SUBCORES
W = 416


@functools.lru_cache(maxsize=None)
def _build(total_n, v):
    chunk = total_n // NUM_WORKERS
    n_win = chunk // W
    mesh = plsc.VectorSubcoreMesh(core_axis_name="c", subcore_axis_name="s")

    n_b = total_n // F
    b_chunk = chunk // F
    b_win = W // F

    @functools.partial(
        pl.kernel,
        out_type=jax.ShapeDtypeStruct((n_b, 32, DP), jnp.float32),
        mesh=mesh,
        scratch_types=[
            pltpu.VMEM((W,), jnp.int32),
            pltpu.VMEM((W,), jnp.int32),
            pltpu.VMEM((W,), jnp.int32),
            pltpu.VMEM((F * D,), jnp.float32),
            pltpu.VMEM((W + 16, DP), jnp.float32),
            pltpu.VMEM((W + 16, DP), jnp.float32),
            pltpu.SemaphoreType.DMA,
            pltpu.SemaphoreType.DMA,
        ],
    )
    def k(table_hbm, x_hbm, offs_hbm, bias_hbm, out_hbm,
          idx0_v, idx1_v, offs_v, bias_v, rows0_v, rows1_v, sem0, sem1):
        wid = lax.axis_index("s") * NUM_CORES + lax.axis_index("c")
        pltpu.sync_copy(offs_hbm, offs_v)
        pltpu.sync_copy(bias_hbm, bias_v)

        def fetch(w, idx_v, rows_v, sem):
            base = wid * chunk + w * W
            pltpu.sync_copy(x_hbm.at[pl.ds(base, W)], idx_v)

            @pl.loop(0, W // LANES)
            def _offset_add(i):
                s = pl.ds(i * LANES, LANES)
                idx_v[s] = idx_v[s] + offs_v[s]

            pltpu.async_copy(table_hbm.at[idx_v],
                             rows_v.at[pl.ds(0, W)], sem)

        def process(w, idx_v, rows_v, sem):
            pltpu.make_async_copy(table_hbm.at[idx_v],
                                  rows_v.at[pl.ds(0, W)], sem).wait()

            @pl.loop(0, F)
            def _bias_field(f):
                b0 = bias_v[pl.ds(f * D, LANES)]
                b1 = bias_v[pl.ds(f * D + LANES, LANES)]

                @pl.loop(0, b_win)
                def _bias_row(g):
                    r = g * F + f
                    rows_v[r, pl.ds(0, LANES)] += b0
                    rows_v[r, pl.ds(LANES, LANES)] += b1

            b_base = wid * b_chunk + w * b_win

            @pl.loop(0, b_win)
            def _store_b(g):
                pltpu.sync_copy(rows_v.at[pl.ds(g * F, 32)],
                                out_hbm.at[b_base + g])

        fetch(0, idx0_v, rows0_v, sem0)

        @pl.loop(0, n_win // 2)
        def _pair(p):
            w0 = 2 * p
            fetch(w0 + 1, idx1_v, rows1_v, sem1)
            process(w0, idx0_v, rows0_v, sem0)

            @pl.when(p < n_win // 2 - 1)
            def _prefetch_even():
                fetch(w0 + 2, idx0_v, rows0_v, sem0)

            process(w0 + 1, idx1_v, rows1_v, sem1)

    return k


_OFFS_TILE = np.tile(np.arange(F, dtype=np.int32) * CARD, W // F)

_TC_LANES = 32768


def _transpose_body(x_ref, o_ref):
    o_ref[:, 0:D] = jnp.transpose(x_ref[...], (1, 0))


@functools.lru_cache(maxsize=None)
def _build_transpose(v):
    grid = (v + _TC_LANES - 1) // _TC_LANES
    return pl.pallas_call(
        _transpose_body,
        grid=(grid,),
        in_specs=[pl.BlockSpec((D, _TC_LANES), lambda i: (0, i))],
        out_specs=pl.BlockSpec((_TC_LANES, DP), lambda i: (i, 0)),
        out_shape=jax.ShapeDtypeStruct((v, DP), jnp.float32),
        compiler_params=pltpu.CompilerParams(
            dimension_semantics=("parallel",)),
    )


def kernel(x, table, bias):
    b, f = x.shape
    n = b * f
    v = table.shape[0]
    x_flat = x.reshape(n)
    offs = jnp.asarray(_OFFS_TILE)
    table_p = _build_transpose(v)(table.T)
    out3 = _build(n, v)(table_p, x_flat, offs, bias.reshape(-1))
    return out3[:, :f, :D]


# speedup vs baseline: 1.9311x; 1.0009x over previous
import functools

import jax
import jax.numpy as jnp
import numpy as np
from jax import lax
from jax.experimental import pallas as pl
from jax.experimental.pallas import tpu as pltpu
from jax.experimental.pallas import tpu_sc as plsc

F = 26
CARD = 100000
D = 32
DP = 128
LANES = 16
NUM_CORES = 2
NUM_SUBCORES = 16
NUM_WORKERS = NUM_CORES * NUM_SUBCORES
W = 416


@functools.lru_cache(maxsize=None)
def _build(total_n, v):
    chunk = total_n // NUM_WORKERS
    n_win = chunk // W
    mesh = plsc.VectorSubcoreMesh(core_axis_name="c", subcore_axis_name="s")

    n_b = total_n // F
    b_chunk = chunk // F
    b_win = W // F

    @functools.partial(
        pl.kernel,
        out_type=jax.ShapeDtypeStruct((n_b, 32, DP), jnp.float32),
        mesh=mesh,
        scratch_types=[
            pltpu.VMEM((W,), jnp.int32),
            pltpu.VMEM((W,), jnp.int32),
            pltpu.VMEM((W,), jnp.int32),
            pltpu.VMEM((F * D,), jnp.float32),
            pltpu.VMEM((W + 16, DP), jnp.float32),
            pltpu.VMEM((W + 16, DP), jnp.float32),
            pltpu.SemaphoreType.DMA,
            pltpu.SemaphoreType.DMA,
        ],
    )
    def k(table_hbm, x_hbm, offs_hbm, bias_hbm, out_hbm,
          idx0_v, idx1_v, offs_v, bias_v, rows0_v, rows1_v, sem0, sem1):
        wid = lax.axis_index("s") * NUM_CORES + lax.axis_index("c")
        pltpu.sync_copy(offs_hbm, offs_v)
        pltpu.sync_copy(bias_hbm, bias_v)

        def fetch(w, idx_v, rows_v, sem):
            base = wid * chunk + w * W
            pltpu.sync_copy(x_hbm.at[pl.ds(base, W)], idx_v)

            @pl.loop(0, W // LANES)
            def _offset_add(i):
                s = pl.ds(i * LANES, LANES)
                idx_v[s] = idx_v[s] + offs_v[s]

            pltpu.async_copy(table_hbm.at[idx_v],
                             rows_v.at[pl.ds(0, W)], sem)

        def process(w, idx_v, rows_v, sem):
            pltpu.make_async_copy(table_hbm.at[idx_v],
                                  rows_v.at[pl.ds(0, W)], sem).wait()

            @pl.loop(0, F)
            def _bias_field(f):
                b0 = bias_v[pl.ds(f * D, LANES)]
                b1 = bias_v[pl.ds(f * D + LANES, LANES)]

                @pl.loop(0, b_win)
                def _bias_row(g):
                    r = g * F + f
                    rows_v[r, pl.ds(0, LANES)] += b0
                    rows_v[r, pl.ds(LANES, LANES)] += b1

            b_base = wid * b_chunk + w * b_win

            @pl.loop(0, b_win)
            def _store_b(g):
                pltpu.sync_copy(rows_v.at[pl.ds(g * F, 32)],
                                out_hbm.at[b_base + g])

        fetch(0, idx0_v, rows0_v, sem0)

        @pl.loop(0, n_win // 2)
        def _pair(p):
            w0 = 2 * p
            fetch(w0 + 1, idx1_v, rows1_v, sem1)
            process(w0, idx0_v, rows0_v, sem0)

            @pl.when(p < n_win // 2 - 1)
            def _prefetch_even():
                fetch(w0 + 2, idx0_v, rows0_v, sem0)

            process(w0 + 1, idx1_v, rows1_v, sem1)

    return k


_OFFS_TILE = np.tile(np.arange(F, dtype=np.int32) * CARD, W // F)

_TC_LANES = 40960


def _transpose_body(x_ref, o_ref):
    o_ref[:, 0:D] = jnp.transpose(x_ref[...], (1, 0))


@functools.lru_cache(maxsize=None)
def _build_transpose(v):
    grid = (v + _TC_LANES - 1) // _TC_LANES
    return pl.pallas_call(
        _transpose_body,
        grid=(grid,),
        in_specs=[pl.BlockSpec((D, _TC_LANES), lambda i: (0, i))],
        out_specs=pl.BlockSpec((_TC_LANES, DP), lambda i: (i, 0)),
        out_shape=jax.ShapeDtypeStruct((v, DP), jnp.float32),
        compiler_params=pltpu.CompilerParams(
            dimension_semantics=("parallel",),
            vmem_limit_bytes=100 * 1024 * 1024),
    )


def kernel(x, table, bias):
    b, f = x.shape
    n = b * f
    v = table.shape[0]
    x_flat = x.reshape(n)
    offs = jnp.asarray(_OFFS_TILE)
    table_p = _build_transpose(v)(table.T)
    out3 = _build(n, v)(table_p, x_flat, offs, bias.reshape(-1))
    return out3[:, :f, :D]
